# baseline (speedup 1.0000x reference)
import functools

import jax
import jax.numpy as jnp
from jax import lax
from jax.experimental import pallas as pl
from jax.experimental.pallas import tpu as pltpu
from jax.experimental.pallas import tpu_sc as plsc

_DIM = 64
_SCALE = 8.0
_LANES = 16
_NC, _NS = 2, 16
_NW = _NC * _NS
_ABLK = 128
_JG = 2


def _gather_scaled(x, idx, embedding):
    A, N1 = x.shape
    B = A * N1
    b_per_w = B // _NW
    na = A // _ABLK // _NW
    ncj = N1 // _JG
    chunk_rows = _JG * _ABLK
    w5_shape = (N1, _DIM // 8, _ABLK, 8, _ABLK)
    mesh = plsc.VectorSubcoreMesh(core_axis_name="c", subcore_axis_name="s")

    @functools.partial(
        pl.kernel,
        out_type=jax.ShapeDtypeStruct(w5_shape, jnp.float32),
        mesh=mesh,
        scratch_types=[
            pltpu.VMEM((b_per_w,), jnp.int32),
            pltpu.VMEM((b_per_w,), jnp.int32),
            pltpu.VMEM((chunk_rows, _DIM), jnp.float32),
            pltpu.VMEM((chunk_rows, _DIM), jnp.float32),
            pltpu.VMEM((_JG, 8, 8, _ABLK), jnp.float32),
            pltpu.VMEM((_JG, 8, 8, _ABLK), jnp.float32),
            pltpu.SemaphoreType.DMA,
            pltpu.SemaphoreType.DMA,
            pltpu.SemaphoreType.DMA,
            pltpu.SemaphoreType.DMA,
        ],
        compiler_params=pltpu.CompilerParams(
            use_tc_tiling_on_sc=False, needs_layout_passes=False),
    )
    def k(emb_hbm, idx_hbm, w5_hbm, idx_v, gidx, g0, g1, t0, t1,
          gs0, gs1, ws0, ws1):
        G = (g0, g1)
        T = (t0, t1)
        gsem = (gs0, gs1)
        wsem = (ws0, ws1)
        wid = lax.axis_index("s") * _NC + lax.axis_index("c")
        base = wid * b_per_w
        pltpu.sync_copy(idx_hbm.at[pl.ds(base, b_per_w)], idx_v)

        iota = lax.iota(jnp.int32, _LANES)
        v50 = iota * N1

        def build_body(cj, carry):
            for ai in range(na):
                for jl in range(_JG):
                    sbase = ai * _ABLK * N1 + cj * _JG + jl
                    dbase = ai * _ABLK * N1 + cj * chunk_rows + jl * _ABLK
                    for s in range(_ABLK // _LANES):
                        v = plsc.load_gather(
                            idx_v, [v50 + (sbase + s * _LANES * N1)])
                        gidx[pl.ds(dbase + s * _LANES, _LANES)] = v
            return carry

        lax.fori_loop(0, ncj, build_body, 0)

        row_vecs = [iota + (jl * _ABLK + s * _LANES)
                    for jl in range(_JG) for s in range(_ABLK // _LANES)]

        def g_desc(ai, cj, b):
            return pltpu.make_async_copy(
                emb_hbm.at[gidx.at[pl.ds(ai * _ABLK * N1 + cj * chunk_rows,
                                         chunk_rows)]],
                G[b], gsem[b])

        def w_desc(ai, cj, b):
            return pltpu.make_async_copy(
                T[b],
                w5_hbm.at[pl.ds(cj * _JG, _JG), :, wid * na + ai],
                wsem[b])

        def compute(ai, b):
            def dd_body(dd, carry):
                dk = lax.shift_right_logical(dd, 3)
                dr = lax.bitwise_and(dd, 7)
                cv = jnp.full((_LANES,), 0, jnp.int32) + dd
                for jl in range(_JG):
                    for s in range(_ABLK // _LANES):
                        v = plsc.load_gather(
                            G[b], [row_vecs[jl * (_ABLK // _LANES) + s], cv])
                        T[b][jl, dk, dr, pl.ds(s * _LANES, _LANES)] = v * _SCALE
                return carry

            lax.fori_loop(0, _DIM, dd_body, 0)

        for ai in range(na):
            g_desc(ai, 0, 0).start()

            def chunk(cj, b):
                g_desc(ai, cj, b).wait()

                @pl.when(cj < ncj - 1)
                def _():
                    g_desc(ai, cj + 1, 1 - b).start()

                @pl.when(cj >= 2)
                def _():
                    w_desc(ai, cj - 2, b).wait()

                compute(ai, b)
                w_desc(ai, cj, b).start()

            def pair_body(g, carry):
                for kk in range(2):
                    chunk(g * 2 + kk, kk)
                return carry

            lax.fori_loop(0, ncj // 2, pair_body, 0)
            chunk(ncj - 1, (ncj - 1) % 2)
            w_desc(ai, ncj - 2, (ncj - 2) % 2).wait()
            w_desc(ai, ncj - 1, (ncj - 1) % 2).wait()

    return k(embedding, idx)


def kernel(x, embedding):
    A, N1 = x.shape
    idx = x.reshape(A * N1).astype(jnp.int32)
    w5 = _gather_scaled(x, idx, embedding)
    out5 = w5.transpose(2, 4, 0, 1, 3)
    return out5.reshape(A, N1, _DIM)


# speedup vs baseline: 1.4787x; 1.4787x over previous
import functools

import jax
import jax.numpy as jnp
from jax import lax
from jax.experimental import pallas as pl
from jax.experimental.pallas import tpu as pltpu
from jax.experimental.pallas import tpu_sc as plsc

_DIM = 64
_SCALE = 8.0
_LANES = 16
_NC, _NS = 2, 16
_NW = _NC * _NS
_ABLK = 128
_JG = 2


def _gather_scaled(x, idx, embedding):
    A, N1 = x.shape
    B = A * N1
    b_per_w = B // _NW
    na = A // _ABLK // _NW
    ncj = N1 // _JG
    chunk_rows = _JG * _ABLK
    w5_shape = (N1, _DIM // 8, _ABLK, 8, _ABLK)
    mesh = plsc.VectorSubcoreMesh(core_axis_name="c", subcore_axis_name="s")

    @functools.partial(
        pl.kernel,
        out_type=jax.ShapeDtypeStruct(w5_shape, jnp.float32),
        mesh=mesh,
        scratch_types=[
            pltpu.VMEM((b_per_w,), jnp.int32),
            pltpu.VMEM((b_per_w,), jnp.int32),
            pltpu.VMEM((chunk_rows, _DIM), jnp.float32),
            pltpu.VMEM((chunk_rows, _DIM), jnp.float32),
            pltpu.VMEM((_JG, 8, 8, _ABLK), jnp.float32),
            pltpu.VMEM((_JG, 8, 8, _ABLK), jnp.float32),
            pltpu.SemaphoreType.DMA,
            pltpu.SemaphoreType.DMA,
            pltpu.SemaphoreType.DMA,
            pltpu.SemaphoreType.DMA,
        ],
        compiler_params=pltpu.CompilerParams(
            use_tc_tiling_on_sc=False, needs_layout_passes=False),
    )
    def k(emb_hbm, idx_hbm, w5_hbm, idx_v, gidx, g0, g1, t0, t1,
          gs0, gs1, ws0, ws1):
        G = (g0, g1)
        T = (t0, t1)
        gsem = (gs0, gs1)
        wsem = (ws0, ws1)
        wid = lax.axis_index("s") * _NC + lax.axis_index("c")
        base = wid * b_per_w
        pltpu.sync_copy(idx_hbm.at[pl.ds(base, b_per_w)], idx_v)

        iota = lax.iota(jnp.int32, _LANES)
        v50 = iota * N1

        def build_body(cj, carry):
            for ai in range(na):
                for jl in range(_JG):
                    sbase = ai * _ABLK * N1 + cj * _JG + jl
                    dbase = ai * _ABLK * N1 + cj * chunk_rows + jl * _ABLK
                    for s in range(_ABLK // _LANES):
                        v = plsc.load_gather(
                            idx_v, [v50 + (sbase + s * _LANES * N1)])
                        gidx[pl.ds(dbase + s * _LANES, _LANES)] = v
            return carry

        lax.fori_loop(0, ncj, build_body, 0)

        dk_lane = lax.shift_right_logical(iota, 3)
        dr_vec = lax.bitwise_and(iota, 7)
        ones = iota * 0 + 1

        def g_desc(ai, cj, b):
            return pltpu.make_async_copy(
                emb_hbm.at[gidx.at[pl.ds(ai * _ABLK * N1 + cj * chunk_rows,
                                         chunk_rows)]],
                G[b], gsem[b])

        def w_desc(ai, cj, b):
            return pltpu.make_async_copy(
                T[b],
                w5_hbm.at[pl.ds(cj * _JG, _JG), :, wid * na + ai],
                wsem[b])

        def compute(ai, b):
            @plsc.parallel_loop(0, chunk_rows, 1, unroll=4)
            def row_body(p):
                jl = lax.shift_right_logical(p, 7)
                al = lax.bitwise_and(p, _ABLK - 1)
                jl_v = ones * jl
                al_v = ones * al
                for d16 in range(_DIM // _LANES):
                    v = G[b][p, pl.ds(d16 * _LANES, _LANES)]
                    plsc.store_scatter(
                        T[b], [jl_v, dk_lane + (d16 * 2), dr_vec, al_v],
                        v * _SCALE)

        for ai in range(na):
            g_desc(ai, 0, 0).start()

            def chunk(cj, b):
                g_desc(ai, cj, b).wait()

                @pl.when(cj < ncj - 1)
                def _():
                    g_desc(ai, cj + 1, 1 - b).start()

                @pl.when(cj >= 2)
                def _():
                    w_desc(ai, cj - 2, b).wait()

                compute(ai, b)
                w_desc(ai, cj, b).start()

            def pair_body(g, carry):
                for kk in range(2):
                    chunk(g * 2 + kk, kk)
                return carry

            lax.fori_loop(0, ncj // 2, pair_body, 0)
            chunk(ncj - 1, (ncj - 1) % 2)
            w_desc(ai, ncj - 2, (ncj - 2) % 2).wait()
            w_desc(ai, ncj - 1, (ncj - 1) % 2).wait()

    return k(embedding, idx)


def kernel(x, embedding):
    A, N1 = x.shape
    idx = x.reshape(A * N1).astype(jnp.int32)
    w5 = _gather_scaled(x, idx, embedding)
    out5 = w5.transpose(2, 4, 0, 1, 3)
    return out5.reshape(A, N1, _DIM)
